# Initial kernel scaffold; baseline (speedup 1.0000x reference)
import jax
import jax.numpy as jnp
from jax.experimental import pallas as pl


def kernel(x, edge_index, W1, b1, W2, b2, predictor, number_of_drugs):
    raise NotImplementedError("write your pallas kernel here")


# speedup vs baseline: 48.0451x; 48.0451x over previous
import functools

import jax
import jax.numpy as jnp
from jax import lax
from jax.experimental import pallas as pl
from jax.experimental.pallas import tpu as pltpu
from jax.experimental.pallas import tpu_sc as plsc

N = 10000
NPAD = 10240
E = 320000
D_FEAT = 128
HID = 16
ND = 2048
NW = 32
C = 128
CH = 80
EPAD = NW * CH * C
SEG = NPAD // 16
JUNK = NPAD - N

_sc_mesh = plsc.VectorSubcoreMesh(core_axis_name="c", subcore_axis_name="s")
_sc_params = pltpu.CompilerParams(use_tc_tiling_on_sc=False)


@functools.partial(
    pl.kernel,
    out_type=jax.ShapeDtypeStruct((2, NPAD), jnp.float32),
    mesh=_sc_mesh,
    compiler_params=_sc_params,
    scratch_types=[
        pltpu.VMEM((CH, C), jnp.int32),
        pltpu.VMEM((C,), jnp.float32),
        pltpu.VMEM((SEG,), jnp.float32),
        pltpu.VMEM_SHARED((NPAD,), jnp.float32),
    ],
)
def _deg_kernel(dst_hbm, out_hbm, dst_v, ones_v, tmp_v, acc):
    cid = lax.axis_index("c")
    sid = lax.axis_index("s")
    wid = sid * 2 + cid

    def _ones(i, _):
        ones_v[pl.ds(i * 16, 16)] = jnp.ones((16,), jnp.float32)
        return 0
    lax.fori_loop(0, C // 16, _ones, 0)

    def _zero(i, _):
        tmp_v[pl.ds(i * 16, 16)] = jnp.zeros((16,), jnp.float32)
        return 0
    lax.fori_loop(0, SEG // 16, _zero, 0)
    pltpu.sync_copy(tmp_v, acc.at[pl.ds(sid * SEG, SEG)])
    pltpu.sync_copy(dst_hbm.at[wid], dst_v)
    plsc.subcore_barrier()

    def _body(j, _):
        pltpu.sync_copy(ones_v, acc.at[dst_v.at[j]], add=True)
        return 0
    lax.fori_loop(0, CH, _body, 0)
    plsc.subcore_barrier()

    pltpu.sync_copy(acc.at[pl.ds(sid * SEG, SEG)], tmp_v)
    pltpu.sync_copy(tmp_v, out_hbm.at[cid, pl.ds(sid * SEG, SEG)])


@functools.partial(
    pl.kernel,
    out_type=jax.ShapeDtypeStruct((2, NPAD, HID), jnp.float32),
    mesh=_sc_mesh,
    compiler_params=_sc_params,
    scratch_types=[
        pltpu.VMEM((CH, C), jnp.int32),
        pltpu.VMEM((CH, C), jnp.int32),
        pltpu.VMEM((2, C, HID), jnp.float32),
        pltpu.VMEM((SEG, HID), jnp.float32),
        pltpu.VMEM_SHARED((NPAD, HID), jnp.float32),
        pltpu.SemaphoreType.DMA,
        pltpu.SemaphoreType.DMA,
    ],
)
def _scatter_kernel(g_hbm, src_hbm, dst_hbm, out_hbm,
                    src_v, dst_v, rows_v, tmp_v, acc, sem0, sem1):
    cid = lax.axis_index("c")
    sid = lax.axis_index("s")
    wid = sid * 2 + cid
    sems = (sem0, sem1)

    def _zero(i, _):
        tmp_v[i, :] = jnp.zeros((16,), jnp.float32)
        return 0
    lax.fori_loop(0, SEG, _zero, 0)
    pltpu.sync_copy(tmp_v, acc.at[pl.ds(sid * SEG, SEG)])
    pltpu.sync_copy(src_hbm.at[wid], src_v)
    pltpu.sync_copy(dst_hbm.at[wid], dst_v)
    plsc.subcore_barrier()

    pltpu.async_copy(g_hbm.at[src_v.at[0]], rows_v.at[0], sem0)
    pltpu.async_copy(g_hbm.at[src_v.at[1]], rows_v.at[1], sem1)

    def _body(i, _):
        for b in range(2):
            j = 2 * i + b
            pltpu.make_async_copy(g_hbm.at[src_v.at[j]], rows_v.at[b],
                                  sems[b]).wait()
            pltpu.sync_copy(rows_v.at[b], acc.at[dst_v.at[j]], add=True)
            pltpu.async_copy(g_hbm.at[src_v.at[j + 2]], rows_v.at[b], sems[b])
        return 0
    lax.fori_loop(0, CH // 2 - 1, _body, 0)
    for b in range(2):
        j = CH - 2 + b
        pltpu.make_async_copy(g_hbm.at[src_v.at[j]], rows_v.at[b],
                              sems[b]).wait()
        pltpu.sync_copy(rows_v.at[b], acc.at[dst_v.at[j]], add=True)
    plsc.subcore_barrier()

    pltpu.sync_copy(acc.at[pl.ds(sid * SEG, SEG)], tmp_v)
    pltpu.sync_copy(tmp_v, out_hbm.at[cid, pl.ds(sid * SEG, SEG)])


def _tc_a_body(x_ref, w1_ref, degp_ref, h_ref, g_ref):
    deg = degp_ref[0, :] + degp_ref[1, :] + 1.0
    dinv = lax.rsqrt(deg)
    h = jnp.dot(x_ref[...], w1_ref[...], preferred_element_type=jnp.float32)
    h_ref[...] = h
    g_ref[...] = h * dinv[:, None]


def _tc_b_body(s1_ref, h_ref, degp_ref, b1_ref, w2_ref, t2_ref, g2_ref):
    deg = degp_ref[0, :] + degp_ref[1, :] + 1.0
    dinv = lax.rsqrt(deg)
    s = s1_ref[0] + s1_ref[1]
    h1 = jnp.maximum(
        s * dinv[:, None] + h_ref[...] * (dinv * dinv)[:, None]
        + b1_ref[0, :][None, :], 0.0)
    t2 = jnp.dot(h1, w2_ref[...], preferred_element_type=jnp.float32)
    t2_ref[...] = t2
    g2_ref[...] = t2 * dinv[:, None]


def _tc_c1_body(s2_ref, t2_ref, degp_ref, b2_ref, pred_ref, d_ref, dp_ref):
    deg = degp_ref[0, :] + degp_ref[1, :] + 1.0
    dinv = lax.rsqrt(deg)
    h2 = ((s2_ref[0] + s2_ref[1]) * dinv[:, None]
          + t2_ref[...] * (dinv * dinv)[:, None] + b2_ref[0, :][None, :])
    d = h2[:ND, :]
    d_ref[...] = d
    dp_ref[...] = jnp.dot(d, pred_ref[...],
                          preferred_element_type=jnp.float32)


def _tc_c2_body(dp_ref, d_ref, o_ref):
    o_ref[...] = lax.dot_general(
        dp_ref[...], d_ref[...], (((1,), (1,)), ((), ())),
        preferred_element_type=jnp.float32)


def kernel(x, edge_index, W1, b1, W2, b2, predictor, number_of_drugs):
    f32 = jnp.float32
    src = edge_index[0]
    dst = edge_index[1]
    pad_rows = N + (jnp.arange(EPAD - E, dtype=jnp.int32) % JUNK)
    src_r = jnp.concatenate([src, pad_rows]).reshape(NW, CH, C)
    dst_r = jnp.concatenate([dst, pad_rows]).reshape(NW, CH, C)
    x_pad = jnp.pad(x, ((0, NPAD - N), (0, 0)))

    degp = _deg_kernel(dst_r)

    h1raw, g1 = pl.pallas_call(
        _tc_a_body,
        out_shape=(jax.ShapeDtypeStruct((NPAD, HID), f32),
                   jax.ShapeDtypeStruct((NPAD, HID), f32)),
    )(x_pad, W1, degp)

    s1 = _scatter_kernel(g1, src_r, dst_r)

    t2, g2 = pl.pallas_call(
        _tc_b_body,
        out_shape=(jax.ShapeDtypeStruct((NPAD, HID), f32),
                   jax.ShapeDtypeStruct((NPAD, HID), f32)),
    )(s1, h1raw, degp, b1.reshape(1, HID), W2)

    s2 = _scatter_kernel(g2, src_r, dst_r)

    d, dp = pl.pallas_call(
        _tc_c1_body,
        out_shape=(jax.ShapeDtypeStruct((ND, HID), f32),
                   jax.ShapeDtypeStruct((ND, HID), f32)),
    )(s2, t2, degp, b2.reshape(1, HID), predictor)

    out = pl.pallas_call(
        _tc_c2_body,
        grid=(8,),
        in_specs=[pl.BlockSpec((ND // 8, HID), lambda i: (i, 0)),
                  pl.BlockSpec((ND, HID), lambda i: (0, 0))],
        out_specs=pl.BlockSpec((ND // 8, ND), lambda i: (i, 0)),
        out_shape=jax.ShapeDtypeStruct((ND, ND), f32),
    )(dp, d)
    return out


# speedup vs baseline: 63.5984x; 1.3237x over previous
import functools

import jax
import jax.numpy as jnp
from jax import lax
from jax.experimental import pallas as pl
from jax.experimental.pallas import tpu as pltpu
from jax.experimental.pallas import tpu_sc as plsc

N = 10000
NPAD = 10240
E = 320000
D_FEAT = 128
HID = 16
ND = 2048
NW = 32
C = 128
CH = 80
EPAD = NW * CH * C
SEG = NPAD // 16
JUNK = NPAD - N

_sc_mesh = plsc.VectorSubcoreMesh(core_axis_name="c", subcore_axis_name="s")
_sc_params = pltpu.CompilerParams(use_tc_tiling_on_sc=False)


@functools.partial(
    pl.kernel,
    out_type=jax.ShapeDtypeStruct((2, NPAD), jnp.float32),
    mesh=_sc_mesh,
    compiler_params=_sc_params,
    scratch_types=[
        pltpu.VMEM((CH, C), jnp.int32),
        pltpu.VMEM((C,), jnp.float32),
        pltpu.VMEM((SEG,), jnp.float32),
        pltpu.VMEM_SHARED((NPAD,), jnp.float32),
        pltpu.SemaphoreType.DMA,
    ],
)
def _deg_kernel(dst_hbm, out_hbm, dst_v, ones_v, tmp_v, acc, sem):
    cid = lax.axis_index("c")
    sid = lax.axis_index("s")
    wid = sid * 2 + cid

    def _ones(i, _):
        ones_v[pl.ds(i * 16, 16)] = jnp.ones((16,), jnp.float32)
        return 0
    lax.fori_loop(0, C // 16, _ones, 0)

    def _zero(i, _):
        tmp_v[pl.ds(i * 16, 16)] = jnp.zeros((16,), jnp.float32)
        return 0
    lax.fori_loop(0, SEG // 16, _zero, 0)
    pltpu.sync_copy(tmp_v, acc.at[pl.ds(sid * SEG, SEG)])
    pltpu.sync_copy(dst_hbm.at[wid], dst_v)
    plsc.subcore_barrier()

    def _grp(i, _):
        for b in range(8):
            pltpu.async_copy(ones_v, acc.at[dst_v.at[8 * i + b]], sem,
                             add=True)
        for b in range(8):
            pltpu.make_async_copy(ones_v, acc.at[dst_v.at[0]], sem).wait()
        return 0
    lax.fori_loop(0, CH // 8, _grp, 0)
    plsc.subcore_barrier()

    pltpu.sync_copy(acc.at[pl.ds(sid * SEG, SEG)], tmp_v)
    pltpu.sync_copy(tmp_v, out_hbm.at[cid, pl.ds(sid * SEG, SEG)])


@functools.partial(
    pl.kernel,
    out_type=jax.ShapeDtypeStruct((2, NPAD, HID), jnp.float32),
    mesh=_sc_mesh,
    compiler_params=_sc_params,
    scratch_types=[
        pltpu.VMEM((CH, C), jnp.int32),
        pltpu.VMEM((CH, C), jnp.int32),
        pltpu.VMEM((8, C, HID), jnp.float32),
        pltpu.VMEM((SEG, HID), jnp.float32),
        pltpu.VMEM_SHARED((NPAD, HID), jnp.float32),
        pltpu.SemaphoreType.DMA((8,)),
        pltpu.SemaphoreType.DMA((8,)),
    ],
)
def _scatter_kernel(g_hbm, src_hbm, dst_hbm, out_hbm,
                    src_v, dst_v, rows_v, tmp_v, acc, sem_g, sem_s):
    cid = lax.axis_index("c")
    sid = lax.axis_index("s")
    wid = sid * 2 + cid

    def _zero(i, _):
        tmp_v[i, :] = jnp.zeros((16,), jnp.float32)
        return 0
    lax.fori_loop(0, C, _zero, 0)
    for k in range(SEG // C):
        pltpu.sync_copy(tmp_v.at[pl.ds(0, C)],
                        acc.at[pl.ds(sid * SEG + k * C, C)])
    pltpu.sync_copy(src_hbm.at[wid], src_v)
    pltpu.sync_copy(dst_hbm.at[wid], dst_v)
    plsc.subcore_barrier()

    for b in range(4):
        pltpu.async_copy(g_hbm.at[src_v.at[b]], rows_v.at[b], sem_g.at[b])

    def _grp(i, _):
        for b in range(8):
            c = 8 * i + b
            bb = (b + 4) % 8
            pltpu.make_async_copy(g_hbm.at[src_v.at[0]], rows_v.at[b],
                                  sem_g.at[b]).wait()
            pltpu.async_copy(rows_v.at[b], acc.at[dst_v.at[c]],
                             sem_s.at[b], add=True)

            @pl.when(c >= 4)
            def _():
                pltpu.make_async_copy(rows_v.at[bb], acc.at[dst_v.at[0]],
                                      sem_s.at[bb]).wait()

            @pl.when(c + 4 < CH)
            def _():
                pltpu.async_copy(g_hbm.at[src_v.at[c + 4]], rows_v.at[bb],
                                 sem_g.at[bb])
        return 0
    lax.fori_loop(0, CH // 8, _grp, 0)
    for b in range(4, 8):
        pltpu.make_async_copy(rows_v.at[b], acc.at[dst_v.at[0]],
                              sem_s.at[b]).wait()
    plsc.subcore_barrier()

    pltpu.sync_copy(acc.at[pl.ds(sid * SEG, SEG)], tmp_v)
    pltpu.sync_copy(tmp_v, out_hbm.at[cid, pl.ds(sid * SEG, SEG)])


def _tc_a_body(x_ref, w1_ref, degp_ref, h_ref, g_ref):
    deg = degp_ref[0, :] + degp_ref[1, :] + 1.0
    dinv = lax.rsqrt(deg)
    h = jnp.dot(x_ref[...], w1_ref[...], preferred_element_type=jnp.float32)
    h_ref[...] = h
    g_ref[...] = h * dinv[:, None]


def _tc_b_body(s1_ref, h_ref, degp_ref, b1_ref, w2_ref, t2_ref, g2_ref):
    deg = degp_ref[0, :] + degp_ref[1, :] + 1.0
    dinv = lax.rsqrt(deg)
    s = s1_ref[0] + s1_ref[1]
    h1 = jnp.maximum(
        s * dinv[:, None] + h_ref[...] * (dinv * dinv)[:, None]
        + b1_ref[0, :][None, :], 0.0)
    t2 = jnp.dot(h1, w2_ref[...], preferred_element_type=jnp.float32)
    t2_ref[...] = t2
    g2_ref[...] = t2 * dinv[:, None]


def _h2_rows(s2_ref, t2_ref, degp_ref, b2_ref):
    deg = degp_ref[0, :] + degp_ref[1, :] + 1.0
    dinv = lax.rsqrt(deg)
    return ((s2_ref[0] + s2_ref[1]) * dinv[:, None]
            + t2_ref[...] * (dinv * dinv)[:, None] + b2_ref[0, :][None, :])


def _tc_c_body(s2b, t2b, dgb, s2f, t2f, dgf, b2_ref, pred_ref, o_ref):
    d_i = _h2_rows(s2b, t2b, dgb, b2_ref)
    d_f = _h2_rows(s2f, t2f, dgf, b2_ref)
    dp_i = jnp.dot(d_i, pred_ref[...], preferred_element_type=jnp.float32)
    o_ref[...] = lax.dot_general(
        dp_i, d_f, (((1,), (1,)), ((), ())),
        preferred_element_type=jnp.float32)


def kernel(x, edge_index, W1, b1, W2, b2, predictor, number_of_drugs):
    f32 = jnp.float32
    src = edge_index[0]
    dst = edge_index[1]
    pad_rows = N + (jnp.arange(EPAD - E, dtype=jnp.int32) % JUNK)
    src_r = jnp.concatenate([src, pad_rows]).reshape(NW, CH, C)
    dst_r = jnp.concatenate([dst, pad_rows]).reshape(NW, CH, C)
    x_pad = jnp.pad(x, ((0, NPAD - N), (0, 0)))

    degp = _deg_kernel(dst_r)

    h1raw, g1 = pl.pallas_call(
        _tc_a_body,
        out_shape=(jax.ShapeDtypeStruct((NPAD, HID), f32),
                   jax.ShapeDtypeStruct((NPAD, HID), f32)),
    )(x_pad, W1, degp)

    s1 = _scatter_kernel(g1, src_r, dst_r)

    t2, g2 = pl.pallas_call(
        _tc_b_body,
        out_shape=(jax.ShapeDtypeStruct((NPAD, HID), f32),
                   jax.ShapeDtypeStruct((NPAD, HID), f32)),
    )(s1, h1raw, degp, b1.reshape(1, HID), W2)

    s2 = _scatter_kernel(g2, src_r, dst_r)

    BR = ND // 8
    out = pl.pallas_call(
        _tc_c_body,
        grid=(8,),
        in_specs=[pl.BlockSpec((2, BR, HID), lambda i: (0, i, 0)),
                  pl.BlockSpec((BR, HID), lambda i: (i, 0)),
                  pl.BlockSpec((2, BR), lambda i: (0, i)),
                  pl.BlockSpec((2, ND, HID), lambda i: (0, 0, 0)),
                  pl.BlockSpec((ND, HID), lambda i: (0, 0)),
                  pl.BlockSpec((2, ND), lambda i: (0, 0)),
                  pl.BlockSpec((1, HID), lambda i: (0, 0)),
                  pl.BlockSpec((HID, HID), lambda i: (0, 0))],
        out_specs=pl.BlockSpec((BR, ND), lambda i: (i, 0)),
        out_shape=jax.ShapeDtypeStruct((ND, ND), f32),
    )(s2, t2, degp, s2, t2, degp, b2.reshape(1, HID), predictor)
    return out


# speedup vs baseline: 64.8260x; 1.0193x over previous
import functools

import jax
import jax.numpy as jnp
from jax import lax
from jax.experimental import pallas as pl
from jax.experimental.pallas import tpu as pltpu
from jax.experimental.pallas import tpu_sc as plsc

N = 10000
NPAD = 10240
E = 320000
D_FEAT = 128
HID = 16
ND = 2048
NW = 32
C = 128
ROWS = E // C
CH = ROWS // NW
XTRA = ROWS - CH * NW
SEG = NPAD // 16
NB = 6

_sc_mesh = plsc.VectorSubcoreMesh(core_axis_name="c", subcore_axis_name="s")
_sc_params = pltpu.CompilerParams(use_tc_tiling_on_sc=False)


@functools.partial(
    pl.kernel,
    out_type=jax.ShapeDtypeStruct((2, NPAD), jnp.float32),
    mesh=_sc_mesh,
    compiler_params=_sc_params,
    scratch_types=[
        pltpu.VMEM((CH, C), jnp.int32),
        pltpu.VMEM((XTRA, C), jnp.int32),
        pltpu.VMEM((C,), jnp.float32),
        pltpu.VMEM((SEG,), jnp.float32),
        pltpu.VMEM_SHARED((NPAD,), jnp.float32),
        pltpu.SemaphoreType.DMA,
    ],
)
def _deg_kernel(ei_hbm, out_hbm, dst_v, xdst_v, ones_v, tmp_v, acc, sem):
    cid = lax.axis_index("c")
    sid = lax.axis_index("s")
    wid = sid * 2 + cid
    base = wid * CH

    def _ones(i, _):
        ones_v[pl.ds(i * 16, 16)] = jnp.ones((16,), jnp.float32)
        return 0
    lax.fori_loop(0, C // 16, _ones, 0)

    def _zero(i, _):
        tmp_v[pl.ds(i * 16, 16)] = jnp.zeros((16,), jnp.float32)
        return 0
    lax.fori_loop(0, SEG // 16, _zero, 0)
    pltpu.sync_copy(tmp_v, acc.at[pl.ds(sid * SEG, SEG)])
    pltpu.sync_copy(ei_hbm.at[1, pl.ds(base, CH)], dst_v)

    @pl.when(wid == NW - 1)
    def _():
        pltpu.sync_copy(ei_hbm.at[1, pl.ds(CH * NW, XTRA)], xdst_v)
    plsc.subcore_barrier()

    def _grp(i, _):
        for b in range(NB):
            pltpu.async_copy(ones_v, acc.at[dst_v.at[NB * i + b]], sem,
                             add=True)
        for b in range(NB):
            pltpu.make_async_copy(ones_v, acc.at[dst_v.at[0]], sem).wait()
        return 0
    lax.fori_loop(0, CH // NB, _grp, 0)

    @pl.when(wid == NW - 1)
    def _():
        for k in range(XTRA):
            pltpu.async_copy(ones_v, acc.at[xdst_v.at[k]], sem, add=True)
        for k in range(XTRA):
            pltpu.make_async_copy(ones_v, acc.at[xdst_v.at[0]], sem).wait()
    plsc.subcore_barrier()

    pltpu.sync_copy(acc.at[pl.ds(sid * SEG, SEG)], tmp_v)
    pltpu.sync_copy(tmp_v, out_hbm.at[cid, pl.ds(sid * SEG, SEG)])


@functools.partial(
    pl.kernel,
    out_type=jax.ShapeDtypeStruct((2, NPAD, HID), jnp.float32),
    mesh=_sc_mesh,
    compiler_params=_sc_params,
    scratch_types=[
        pltpu.VMEM((CH, C), jnp.int32),
        pltpu.VMEM((CH, C), jnp.int32),
        pltpu.VMEM((XTRA, C), jnp.int32),
        pltpu.VMEM((XTRA, C), jnp.int32),
        pltpu.VMEM((NB, C, HID), jnp.float32),
        pltpu.VMEM((SEG, HID), jnp.float32),
        pltpu.VMEM_SHARED((NPAD, HID), jnp.float32),
        pltpu.SemaphoreType.DMA((NB,)),
        pltpu.SemaphoreType.DMA((NB,)),
    ],
)
def _scatter_kernel(g_hbm, ei_hbm, out_hbm, src_v, dst_v, xsrc_v, xdst_v,
                    rows_v, tmp_v, acc, sem_g, sem_s):
    cid = lax.axis_index("c")
    sid = lax.axis_index("s")
    wid = sid * 2 + cid
    base = wid * CH

    def _zero(i, _):
        tmp_v[i, :] = jnp.zeros((16,), jnp.float32)
        return 0
    lax.fori_loop(0, C, _zero, 0)
    for k in range(SEG // C):
        pltpu.sync_copy(tmp_v.at[pl.ds(0, C)],
                        acc.at[pl.ds(sid * SEG + k * C, C)])
    pltpu.sync_copy(ei_hbm.at[0, pl.ds(base, CH)], src_v)
    pltpu.sync_copy(ei_hbm.at[1, pl.ds(base, CH)], dst_v)

    @pl.when(wid == NW - 1)
    def _():
        pltpu.sync_copy(ei_hbm.at[0, pl.ds(CH * NW, XTRA)], xsrc_v)
        pltpu.sync_copy(ei_hbm.at[1, pl.ds(CH * NW, XTRA)], xdst_v)
    plsc.subcore_barrier()

    for b in range(NB // 2):
        pltpu.async_copy(g_hbm.at[src_v.at[b]], rows_v.at[b], sem_g.at[b])

    def _grp(i, _):
        for b in range(NB):
            c = NB * i + b
            bb = (b + 3) % NB
            pltpu.make_async_copy(g_hbm.at[src_v.at[0]], rows_v.at[b],
                                  sem_g.at[b]).wait()
            pltpu.async_copy(rows_v.at[b], acc.at[dst_v.at[c]],
                             sem_s.at[b], add=True)

            @pl.when(c >= 3)
            def _():
                pltpu.make_async_copy(rows_v.at[bb], acc.at[dst_v.at[0]],
                                      sem_s.at[bb]).wait()

            @pl.when(c + 3 < CH)
            def _():
                pltpu.async_copy(g_hbm.at[src_v.at[c + 3]], rows_v.at[bb],
                                 sem_g.at[bb])
        return 0
    lax.fori_loop(0, CH // NB, _grp, 0)

    @pl.when(wid == NW - 1)
    def _():
        for k in range(3):
            pltpu.async_copy(g_hbm.at[xsrc_v.at[k]], rows_v.at[k],
                             sem_g.at[k])
        for k in range(3):
            pltpu.make_async_copy(g_hbm.at[xsrc_v.at[0]], rows_v.at[k],
                                  sem_g.at[k]).wait()
            pltpu.async_copy(rows_v.at[k], acc.at[xdst_v.at[k]],
                             sem_s.at[k], add=True)
        pltpu.make_async_copy(rows_v.at[3], acc.at[xdst_v.at[0]],
                              sem_s.at[3]).wait()
        pltpu.async_copy(g_hbm.at[xsrc_v.at[3]], rows_v.at[3], sem_g.at[3])
        pltpu.make_async_copy(g_hbm.at[xsrc_v.at[0]], rows_v.at[3],
                              sem_g.at[3]).wait()
        pltpu.async_copy(rows_v.at[3], acc.at[xdst_v.at[3]],
                         sem_s.at[3], add=True)
        for k in range(NB):
            pltpu.make_async_copy(rows_v.at[k], acc.at[xdst_v.at[0]],
                                  sem_s.at[k]).wait()

    @pl.when(wid != NW - 1)
    def _():
        for k in range(3, NB):
            pltpu.make_async_copy(rows_v.at[k], acc.at[dst_v.at[0]],
                                  sem_s.at[k]).wait()
    plsc.subcore_barrier()

    pltpu.sync_copy(acc.at[pl.ds(sid * SEG, SEG)], tmp_v)
    pltpu.sync_copy(tmp_v, out_hbm.at[cid, pl.ds(sid * SEG, SEG)])


def _tc_a_body(x_ref, w1_ref, degp_ref, h_ref, g_ref):
    degp = degp_ref[...]
    deg = degp[0, :N] + degp[1, :N] + 1.0
    dinv = lax.rsqrt(deg)
    h = jnp.dot(x_ref[...], w1_ref[...], preferred_element_type=jnp.float32)
    h_ref[...] = h
    g_ref[...] = h * dinv[:, None]


def _tc_b_body(s1_ref, h_ref, degp_ref, b1_ref, w2_ref, t2_ref, g2_ref):
    degp = degp_ref[...]
    deg = degp[0, :N] + degp[1, :N] + 1.0
    dinv = lax.rsqrt(deg)
    s = s1_ref[0, :N, :] + s1_ref[1, :N, :]
    h1 = jnp.maximum(
        s * dinv[:, None] + h_ref[...] * (dinv * dinv)[:, None]
        + b1_ref[0, :][None, :], 0.0)
    t2 = jnp.dot(h1, w2_ref[...], preferred_element_type=jnp.float32)
    t2_ref[...] = t2
    g2_ref[...] = t2 * dinv[:, None]


def _h2_rows(s2, t2, degp, b2):
    deg = degp[0, :] + degp[1, :] + 1.0
    dinv = lax.rsqrt(deg)
    return (s2[0] + s2[1]) * dinv[:, None] \
        + t2 * (dinv * dinv)[:, None] + b2[0, :][None, :]


def _tc_c_body(s2b, t2b, dgb, s2f, t2f, dgf, b2_ref, pred_ref, o_ref):
    b2 = b2_ref[...]
    d_i = _h2_rows(s2b[...], t2b[...], dgb[...], b2)
    d_f = _h2_rows(s2f[...], t2f[...], dgf[...], b2)
    dp_i = jnp.dot(d_i, pred_ref[...], preferred_element_type=jnp.float32)
    o_ref[...] = lax.dot_general(
        dp_i, d_f, (((1,), (1,)), ((), ())),
        preferred_element_type=jnp.float32)


def kernel(x, edge_index, W1, b1, W2, b2, predictor, number_of_drugs):
    f32 = jnp.float32
    ei_r = edge_index.reshape(2, ROWS, C)

    degp = _deg_kernel(ei_r)

    h1raw, g1 = pl.pallas_call(
        _tc_a_body,
        out_shape=(jax.ShapeDtypeStruct((N, HID), f32),
                   jax.ShapeDtypeStruct((N, HID), f32)),
    )(x, W1, degp)

    s1 = _scatter_kernel(g1, ei_r)

    t2, g2 = pl.pallas_call(
        _tc_b_body,
        out_shape=(jax.ShapeDtypeStruct((N, HID), f32),
                   jax.ShapeDtypeStruct((N, HID), f32)),
    )(s1, h1raw, degp, b1.reshape(1, HID), W2)

    s2 = _scatter_kernel(g2, ei_r)

    BR = ND // 8
    out = pl.pallas_call(
        _tc_c_body,
        grid=(8,),
        in_specs=[pl.BlockSpec((2, BR, HID), lambda i: (0, i, 0)),
                  pl.BlockSpec((BR, HID), lambda i: (i, 0)),
                  pl.BlockSpec((2, BR), lambda i: (0, i)),
                  pl.BlockSpec((2, ND, HID), lambda i: (0, 0, 0)),
                  pl.BlockSpec((ND, HID), lambda i: (0, 0)),
                  pl.BlockSpec((2, ND), lambda i: (0, 0)),
                  pl.BlockSpec((1, HID), lambda i: (0, 0)),
                  pl.BlockSpec((HID, HID), lambda i: (0, 0))],
        out_specs=pl.BlockSpec((BR, ND), lambda i: (i, 0)),
        out_shape=jax.ShapeDtypeStruct((ND, ND), f32),
    )(s2, t2, degp, s2, t2, degp, b2.reshape(1, HID), predictor)
    return out


# speedup vs baseline: 67.8934x; 1.0473x over previous
import functools

import jax
import jax.numpy as jnp
from jax import lax
from jax.experimental import pallas as pl
from jax.experimental.pallas import tpu as pltpu
from jax.experimental.pallas import tpu_sc as plsc

N = 10000
NPAD = 10240
E = 320000
D_FEAT = 128
HID = 16
ND = 2048
NW = 32
C = 128
ROWS = E // C
CH = ROWS // NW
XTRA = ROWS - CH * NW
SEG = NPAD // 16
NB = 8
CHM = (CH // NB) * NB

_sc_mesh = plsc.VectorSubcoreMesh(core_axis_name="c", subcore_axis_name="s")
_sc_params = pltpu.CompilerParams(use_tc_tiling_on_sc=False)


@functools.partial(
    pl.kernel,
    out_type=jax.ShapeDtypeStruct((2, NPAD), jnp.float32),
    mesh=_sc_mesh,
    compiler_params=_sc_params,
    scratch_types=[
        pltpu.VMEM((CH, C), jnp.int32),
        pltpu.VMEM((XTRA, C), jnp.int32),
        pltpu.VMEM((C,), jnp.float32),
        pltpu.VMEM((SEG,), jnp.float32),
        pltpu.VMEM_SHARED((NPAD,), jnp.float32),
        pltpu.SemaphoreType.DMA,
    ],
)
def _deg_kernel(ei_hbm, out_hbm, dst_v, xdst_v, ones_v, tmp_v, acc, sem):
    cid = lax.axis_index("c")
    sid = lax.axis_index("s")
    wid = sid * 2 + cid
    base = wid * CH

    def _ones(i, _):
        ones_v[pl.ds(i * 16, 16)] = jnp.ones((16,), jnp.float32)
        return 0
    lax.fori_loop(0, C // 16, _ones, 0)

    def _zero(i, _):
        tmp_v[pl.ds(i * 16, 16)] = jnp.zeros((16,), jnp.float32)
        return 0
    lax.fori_loop(0, SEG // 16, _zero, 0)
    pltpu.sync_copy(tmp_v, acc.at[pl.ds(sid * SEG, SEG)])
    pltpu.sync_copy(ei_hbm.at[1, pl.ds(base, CH)], dst_v)

    @pl.when(wid == NW - 1)
    def _():
        pltpu.sync_copy(ei_hbm.at[1, pl.ds(CH * NW, XTRA)], xdst_v)
    plsc.subcore_barrier()

    DG = 6
    def _grp(i, _):
        for b in range(DG):
            pltpu.async_copy(ones_v, acc.at[dst_v.at[DG * i + b]], sem,
                             add=True)
        for b in range(DG):
            pltpu.make_async_copy(ones_v, acc.at[dst_v.at[0]], sem).wait()
        return 0
    lax.fori_loop(0, CH // DG, _grp, 0)

    @pl.when(wid == NW - 1)
    def _():
        for k in range(XTRA):
            pltpu.async_copy(ones_v, acc.at[xdst_v.at[k]], sem, add=True)
        for k in range(XTRA):
            pltpu.make_async_copy(ones_v, acc.at[xdst_v.at[0]], sem).wait()
    plsc.subcore_barrier()

    pltpu.sync_copy(acc.at[pl.ds(sid * SEG, SEG)], tmp_v)
    pltpu.sync_copy(tmp_v, out_hbm.at[cid, pl.ds(sid * SEG, SEG)])


@functools.partial(
    pl.kernel,
    out_type=jax.ShapeDtypeStruct((2, NPAD, HID), jnp.float32),
    mesh=_sc_mesh,
    compiler_params=_sc_params,
    scratch_types=[
        pltpu.VMEM((CH, C), jnp.int32),
        pltpu.VMEM((CH, C), jnp.int32),
        pltpu.VMEM((XTRA, C), jnp.int32),
        pltpu.VMEM((XTRA, C), jnp.int32),
        pltpu.VMEM((NB, C, HID), jnp.float32),
        pltpu.VMEM((SEG, HID), jnp.float32),
        pltpu.VMEM_SHARED((NPAD, HID), jnp.float32),
        pltpu.SemaphoreType.DMA((NB,)),
        pltpu.SemaphoreType.DMA((NB,)),
    ],
)
def _scatter_kernel(g_hbm, ei_hbm, out_hbm, src_v, dst_v, xsrc_v, xdst_v,
                    rows_v, tmp_v, acc, sem_g, sem_s):
    cid = lax.axis_index("c")
    sid = lax.axis_index("s")
    wid = sid * 2 + cid
    base = wid * CH

    def _zero(i, _):
        tmp_v[i, :] = jnp.zeros((16,), jnp.float32)
        return 0
    lax.fori_loop(0, C, _zero, 0)
    for k in range(SEG // C):
        pltpu.sync_copy(tmp_v.at[pl.ds(0, C)],
                        acc.at[pl.ds(sid * SEG + k * C, C)])
    pltpu.sync_copy(ei_hbm.at[0, pl.ds(base, CH)], src_v)
    pltpu.sync_copy(ei_hbm.at[1, pl.ds(base, CH)], dst_v)

    @pl.when(wid == NW - 1)
    def _():
        pltpu.sync_copy(ei_hbm.at[0, pl.ds(CH * NW, XTRA)], xsrc_v)
        pltpu.sync_copy(ei_hbm.at[1, pl.ds(CH * NW, XTRA)], xdst_v)
    plsc.subcore_barrier()

    def _wait_g(b):
        pltpu.make_async_copy(g_hbm.at[src_v.at[0]], rows_v.at[b],
                              sem_g.at[b]).wait()

    def _wait_s(b):
        pltpu.make_async_copy(rows_v.at[b], acc.at[dst_v.at[0]],
                              sem_s.at[b]).wait()

    for b in range(NB // 2):
        pltpu.async_copy(g_hbm.at[src_v.at[b]], rows_v.at[b], sem_g.at[b])

    def _grp(i, _):
        for b in range(NB):
            c = NB * i + b
            bb = (b + 4) % NB
            _wait_g(b)
            pltpu.async_copy(rows_v.at[b], acc.at[dst_v.at[c]],
                             sem_s.at[b], add=True)

            @pl.when(c >= 4)
            def _():
                _wait_s(bb)

            @pl.when(c + 4 < CH)
            def _():
                pltpu.async_copy(g_hbm.at[src_v.at[c + 4]], rows_v.at[bb],
                                 sem_g.at[bb])
        return 0
    lax.fori_loop(0, CHM // NB, _grp, 0)

    for c in range(CHM, CH):
        b = c % NB
        bb = (b + 4) % NB
        _wait_g(b)
        pltpu.async_copy(rows_v.at[b], acc.at[dst_v.at[c]],
                         sem_s.at[b], add=True)
        _wait_s(bb)
        if c + 4 < CH:
            pltpu.async_copy(g_hbm.at[src_v.at[c + 4]], rows_v.at[bb],
                             sem_g.at[bb])
    @pl.when(wid == NW - 1)
    def _():
        xb = (6, 7, 0, 1)
        for k in range(XTRA):
            pltpu.async_copy(g_hbm.at[xsrc_v.at[k]], rows_v.at[xb[k]],
                             sem_g.at[xb[k]])
        for k in range(XTRA):
            pltpu.make_async_copy(g_hbm.at[xsrc_v.at[0]], rows_v.at[xb[k]],
                                  sem_g.at[xb[k]]).wait()
            pltpu.async_copy(rows_v.at[xb[k]], acc.at[xdst_v.at[k]],
                             sem_s.at[xb[k]], add=True)
        for b in range(NB):
            _wait_s(b)

    @pl.when(wid != NW - 1)
    def _():
        for b in range(2, 6):
            _wait_s(b)
    plsc.subcore_barrier()

    pltpu.sync_copy(acc.at[pl.ds(sid * SEG, SEG)], tmp_v)
    pltpu.sync_copy(tmp_v, out_hbm.at[cid, pl.ds(sid * SEG, SEG)])


def _dinv_block(degp_ref):
    degp = degp_ref[...]
    deg = degp[0, :] + degp[1, :] + 1.0
    return lax.rsqrt(deg)


def _tc_a_body(x_ref, w1_ref, degp_ref, h_ref, g_ref):
    dinv = _dinv_block(degp_ref)
    h = jnp.dot(x_ref[...], w1_ref[...], preferred_element_type=jnp.float32)
    h_ref[...] = h
    g_ref[...] = h * dinv[:, None]


def _tc_b_body(s1_ref, h_ref, degp_ref, b1_ref, w2_ref, t2_ref, g2_ref):
    dinv = _dinv_block(degp_ref)
    s = s1_ref[0] + s1_ref[1]
    h1 = jnp.maximum(
        s * dinv[:, None] + h_ref[...] * (dinv * dinv)[:, None]
        + b1_ref[0, :][None, :], 0.0)
    t2 = jnp.dot(h1, w2_ref[...], preferred_element_type=jnp.float32)
    t2_ref[...] = t2
    g2_ref[...] = t2 * dinv[:, None]


def _h2_rows(s2, t2, degp, b2):
    deg = degp[0, :] + degp[1, :] + 1.0
    dinv = lax.rsqrt(deg)
    return (s2[0] + s2[1]) * dinv[:, None] \
        + t2 * (dinv * dinv)[:, None] + b2[0, :][None, :]


def _tc_c_body(s2b, t2b, dgb, s2f, t2f, dgf, b2_ref, pred_ref, o_ref):
    b2 = b2_ref[...]
    d_i = _h2_rows(s2b[...], t2b[...], dgb[...], b2)
    d_f = _h2_rows(s2f[...], t2f[...], dgf[...], b2)
    dp_i = jnp.dot(d_i, pred_ref[...], preferred_element_type=jnp.float32)
    o_ref[...] = lax.dot_general(
        dp_i, d_f, (((1,), (1,)), ((), ())),
        preferred_element_type=jnp.float32)


def kernel(x, edge_index, W1, b1, W2, b2, predictor, number_of_drugs):
    f32 = jnp.float32
    ei_r = edge_index.reshape(2, ROWS, C)

    degp = _deg_kernel(ei_r)

    x_pad = jnp.pad(x, ((0, NPAD - N), (0, 0)))
    RB = 2048
    h1raw, g1 = pl.pallas_call(
        _tc_a_body,
        grid=(NPAD // RB,),
        in_specs=[pl.BlockSpec((RB, D_FEAT), lambda i: (i, 0)),
                  pl.BlockSpec((D_FEAT, HID), lambda i: (0, 0)),
                  pl.BlockSpec((2, RB), lambda i: (0, i))],
        out_specs=(pl.BlockSpec((RB, HID), lambda i: (i, 0)),
                   pl.BlockSpec((RB, HID), lambda i: (i, 0))),
        out_shape=(jax.ShapeDtypeStruct((NPAD, HID), f32),
                   jax.ShapeDtypeStruct((NPAD, HID), f32)),
    )(x_pad, W1, degp)

    s1 = _scatter_kernel(g1, ei_r)

    t2, g2 = pl.pallas_call(
        _tc_b_body,
        grid=(NPAD // RB,),
        in_specs=[pl.BlockSpec((2, RB, HID), lambda i: (0, i, 0)),
                  pl.BlockSpec((RB, HID), lambda i: (i, 0)),
                  pl.BlockSpec((2, RB), lambda i: (0, i)),
                  pl.BlockSpec((1, HID), lambda i: (0, 0)),
                  pl.BlockSpec((HID, HID), lambda i: (0, 0))],
        out_specs=(pl.BlockSpec((RB, HID), lambda i: (i, 0)),
                   pl.BlockSpec((RB, HID), lambda i: (i, 0))),
        out_shape=(jax.ShapeDtypeStruct((NPAD, HID), f32),
                   jax.ShapeDtypeStruct((NPAD, HID), f32)),
    )(s1, h1raw, degp, b1.reshape(1, HID), W2)

    s2 = _scatter_kernel(g2, ei_r)

    BR = ND // 8
    out = pl.pallas_call(
        _tc_c_body,
        grid=(8,),
        in_specs=[pl.BlockSpec((2, BR, HID), lambda i: (0, i, 0)),
                  pl.BlockSpec((BR, HID), lambda i: (i, 0)),
                  pl.BlockSpec((2, BR), lambda i: (0, i)),
                  pl.BlockSpec((2, ND, HID), lambda i: (0, 0, 0)),
                  pl.BlockSpec((ND, HID), lambda i: (0, 0)),
                  pl.BlockSpec((2, ND), lambda i: (0, 0)),
                  pl.BlockSpec((1, HID), lambda i: (0, 0)),
                  pl.BlockSpec((HID, HID), lambda i: (0, 0))],
        out_specs=pl.BlockSpec((BR, ND), lambda i: (i, 0)),
        out_shape=jax.ShapeDtypeStruct((ND, ND), f32),
    )(s2, t2, degp, s2, t2, degp, b2.reshape(1, HID), predictor)
    return out


# speedup vs baseline: 81.1575x; 1.1954x over previous
import functools

import jax
import jax.numpy as jnp
from jax import lax
from jax.experimental import pallas as pl
from jax.experimental.pallas import tpu as pltpu
from jax.experimental.pallas import tpu_sc as plsc

N = 10000
NPAD = 10240
E = 320000
D_FEAT = 128
HID = 16
ND = 2048
NW = 32
C = 128
ROWS = E // C
CH = ROWS // NW
XTRA = ROWS - CH * NW
SEG = NPAD // 16
NB = 8
CHM = (CH // NB) * NB

_sc_mesh = plsc.VectorSubcoreMesh(core_axis_name="c", subcore_axis_name="s")
_sc_params = pltpu.CompilerParams(use_tc_tiling_on_sc=False)


@functools.partial(
    pl.kernel,
    out_type=jax.ShapeDtypeStruct((2, NPAD), jnp.float32),
    mesh=_sc_mesh,
    compiler_params=_sc_params,
    scratch_types=[
        pltpu.VMEM((CH, C), jnp.int32),
        pltpu.VMEM((XTRA, C), jnp.int32),
        pltpu.VMEM((C,), jnp.float32),
        pltpu.VMEM((SEG,), jnp.float32),
        pltpu.VMEM_SHARED((NPAD,), jnp.float32),
        pltpu.SemaphoreType.DMA,
    ],
)
def _deg_kernel(ei_hbm, out_hbm, dst_v, xdst_v, ones_v, tmp_v, acc, sem):
    cid = lax.axis_index("c")
    sid = lax.axis_index("s")
    wid = sid * 2 + cid
    base = wid * CH

    def _ones(i, _):
        ones_v[pl.ds(i * 16, 16)] = jnp.ones((16,), jnp.float32)
        return 0
    lax.fori_loop(0, C // 16, _ones, 0)

    def _zero(i, _):
        tmp_v[pl.ds(i * 16, 16)] = jnp.zeros((16,), jnp.float32)
        return 0
    lax.fori_loop(0, SEG // 16, _zero, 0)
    pltpu.sync_copy(tmp_v, acc.at[pl.ds(sid * SEG, SEG)])
    pltpu.sync_copy(ei_hbm.at[1, pl.ds(base, CH)], dst_v)

    @pl.when(wid == NW - 1)
    def _():
        pltpu.sync_copy(ei_hbm.at[1, pl.ds(CH * NW, XTRA)], xdst_v)
    plsc.subcore_barrier()

    DG = 6
    def _grp(i, _):
        for b in range(DG):
            pltpu.async_copy(ones_v, acc.at[dst_v.at[DG * i + b]], sem,
                             add=True)
        for b in range(DG):
            pltpu.make_async_copy(ones_v, acc.at[dst_v.at[0]], sem).wait()
        return 0
    lax.fori_loop(0, CH // DG, _grp, 0)

    @pl.when(wid == NW - 1)
    def _():
        for k in range(XTRA):
            pltpu.async_copy(ones_v, acc.at[xdst_v.at[k]], sem, add=True)
        for k in range(XTRA):
            pltpu.make_async_copy(ones_v, acc.at[xdst_v.at[0]], sem).wait()
    plsc.subcore_barrier()

    pltpu.sync_copy(acc.at[pl.ds(sid * SEG, SEG)], tmp_v)
    pltpu.sync_copy(tmp_v, out_hbm.at[cid, pl.ds(sid * SEG, SEG)])


@functools.partial(
    pl.kernel,
    out_type=jax.ShapeDtypeStruct((2, NPAD, HID), jnp.float32),
    mesh=_sc_mesh,
    compiler_params=_sc_params,
    scratch_types=[
        pltpu.VMEM((CH, C), jnp.int32),
        pltpu.VMEM((CH, C), jnp.int32),
        pltpu.VMEM((XTRA, C), jnp.int32),
        pltpu.VMEM((XTRA, C), jnp.int32),
        pltpu.VMEM((NB, C, HID), jnp.float32),
        pltpu.VMEM((SEG, HID), jnp.float32),
        pltpu.VMEM_SHARED((NPAD, HID), jnp.float32),
        pltpu.SemaphoreType.DMA((NB,)),
        pltpu.SemaphoreType.DMA((NB,)),
    ],
)
def _scatter_kernel(g_hbm, ei_hbm, out_hbm, src_v, dst_v, xsrc_v, xdst_v,
                    rows_v, tmp_v, acc, sem_g, sem_s):
    cid = lax.axis_index("c")
    sid = lax.axis_index("s")
    wid = sid * 2 + cid
    base = wid * CH

    def _zero(i, _):
        tmp_v[i, :] = jnp.zeros((16,), jnp.float32)
        return 0
    lax.fori_loop(0, C, _zero, 0)
    for k in range(SEG // C):
        pltpu.sync_copy(tmp_v.at[pl.ds(0, C)],
                        acc.at[pl.ds(sid * SEG + k * C, C)])
    pltpu.sync_copy(ei_hbm.at[0, pl.ds(base, CH)], src_v)
    pltpu.sync_copy(ei_hbm.at[1, pl.ds(base, CH)], dst_v)

    @pl.when(wid == NW - 1)
    def _():
        pltpu.sync_copy(ei_hbm.at[0, pl.ds(CH * NW, XTRA)], xsrc_v)
        pltpu.sync_copy(ei_hbm.at[1, pl.ds(CH * NW, XTRA)], xdst_v)
    plsc.subcore_barrier()

    def _wait_g(b):
        pltpu.make_async_copy(g_hbm.at[src_v.at[0]], rows_v.at[b],
                              sem_g.at[b]).wait()

    def _wait_s(b):
        pltpu.make_async_copy(rows_v.at[b], acc.at[dst_v.at[0]],
                              sem_s.at[b]).wait()

    for b in range(NB // 2):
        pltpu.async_copy(g_hbm.at[src_v.at[b]], rows_v.at[b], sem_g.at[b])

    def _grp(i, _):
        for b in range(NB):
            c = NB * i + b
            bb = (b + 4) % NB
            _wait_g(b)
            pltpu.async_copy(rows_v.at[b], acc.at[dst_v.at[c]],
                             sem_s.at[b], add=True)

            @pl.when(c >= 4)
            def _():
                _wait_s(bb)

            @pl.when(c + 4 < CH)
            def _():
                pltpu.async_copy(g_hbm.at[src_v.at[c + 4]], rows_v.at[bb],
                                 sem_g.at[bb])
        return 0
    lax.fori_loop(0, CHM // NB, _grp, 0)

    for c in range(CHM, CH):
        b = c % NB
        bb = (b + 4) % NB
        _wait_g(b)
        pltpu.async_copy(rows_v.at[b], acc.at[dst_v.at[c]],
                         sem_s.at[b], add=True)
        _wait_s(bb)
        if c + 4 < CH:
            pltpu.async_copy(g_hbm.at[src_v.at[c + 4]], rows_v.at[bb],
                             sem_g.at[bb])
    @pl.when(wid == NW - 1)
    def _():
        xb = (6, 7, 0, 1)
        for k in range(XTRA):
            pltpu.async_copy(g_hbm.at[xsrc_v.at[k]], rows_v.at[xb[k]],
                             sem_g.at[xb[k]])
        for k in range(XTRA):
            pltpu.make_async_copy(g_hbm.at[xsrc_v.at[0]], rows_v.at[xb[k]],
                                  sem_g.at[xb[k]]).wait()
            pltpu.async_copy(rows_v.at[xb[k]], acc.at[xdst_v.at[k]],
                             sem_s.at[xb[k]], add=True)
        for b in range(NB):
            _wait_s(b)

    @pl.when(wid != NW - 1)
    def _():
        for b in range(2, 6):
            _wait_s(b)
    plsc.subcore_barrier()

    pltpu.sync_copy(acc.at[pl.ds(sid * SEG, SEG)], tmp_v)
    pltpu.sync_copy(tmp_v, out_hbm.at[cid, pl.ds(sid * SEG, SEG)])


def _tc_a_body(xp_ref, w1_ref, deg8_ref, rep_ref, h_ref, g_ref, dv_ref):
    d8 = deg8_ref[0] + deg8_ref[1] + 1.0
    dv = lax.rsqrt(jnp.dot(d8, rep_ref[...],
                           preferred_element_type=jnp.float32))
    xp = xp_ref[...]
    w1 = w1_ref[...]
    h = jnp.concatenate(
        [jnp.dot(xp[:, 128 * a:128 * (a + 1)], w1,
                 preferred_element_type=jnp.float32) for a in range(8)],
        axis=1)
    h_ref[...] = h
    g_ref[...] = h * dv
    dv_ref[...] = dv


def _tc_b_body(s1_ref, h_ref, dv_ref, b1t_ref, w2_ref, t2_ref, g2_ref):
    dv = dv_ref[...]
    w2 = w2_ref[...]
    s = s1_ref[0] + s1_ref[1]
    h1 = jnp.maximum(
        s * dv + h_ref[...] * dv * dv + b1t_ref[0, :][None, :], 0.0)
    t2 = jnp.concatenate(
        [jnp.dot(h1[:, 16 * a:16 * (a + 1)], w2,
                 preferred_element_type=jnp.float32) for a in range(8)],
        axis=1)
    t2_ref[...] = t2
    g2_ref[...] = t2 * dv


def _h2_rows(s2, t2, degp, b2):
    deg = degp[0, :] + degp[1, :] + 1.0
    dinv = lax.rsqrt(deg)
    return (s2[0] + s2[1]) * dinv[:, None] \
        + t2 * (dinv * dinv)[:, None] + b2[0, :][None, :]


def _tc_c_body(s2b, t2b, dgb, s2f, t2f, dgf, b2_ref, pred_ref, o_ref):
    b2 = b2_ref[...]
    d_i = _h2_rows(s2b[...], t2b[...], dgb[...], b2)
    d_f = _h2_rows(s2f[...], t2f[...], dgf[...], b2)
    dp_i = jnp.dot(d_i, pred_ref[...], preferred_element_type=jnp.float32)
    o_ref[...] = lax.dot_general(
        dp_i, d_f, (((1,), (1,)), ((), ())),
        preferred_element_type=jnp.float32)


def kernel(x, edge_index, W1, b1, W2, b2, predictor, number_of_drugs):
    f32 = jnp.float32
    ei_r = edge_index.reshape(2, ROWS, C)

    degp = _deg_kernel(ei_r)

    x_pad = jnp.pad(x, ((0, NPAD - N), (0, 0)))
    RB = 2048
    PR = NPAD // 8
    PB = RB // 8
    PK = 128
    xp = x_pad.reshape(PR, 8 * D_FEAT)
    rep = jnp.repeat(jnp.eye(8, dtype=f32), HID, axis=1)
    deg8 = degp.reshape(2, PR, 8)
    h1p, g1p, dvp = pl.pallas_call(
        _tc_a_body,
        grid=(NPAD // RB,),
        in_specs=[pl.BlockSpec((PB, 8 * D_FEAT), lambda i: (i, 0)),
                  pl.BlockSpec((D_FEAT, HID), lambda i: (0, 0)),
                  pl.BlockSpec((2, PB, 8), lambda i: (0, i, 0)),
                  pl.BlockSpec((8, PK), lambda i: (0, 0))],
        out_specs=(pl.BlockSpec((PB, PK), lambda i: (i, 0)),
                   pl.BlockSpec((PB, PK), lambda i: (i, 0)),
                   pl.BlockSpec((PB, PK), lambda i: (i, 0))),
        out_shape=(jax.ShapeDtypeStruct((PR, PK), f32),
                   jax.ShapeDtypeStruct((PR, PK), f32),
                   jax.ShapeDtypeStruct((PR, PK), f32)),
    )(xp, W1, deg8, rep)

    s1 = _scatter_kernel(g1p.reshape(NPAD, HID), ei_r)

    t2p, g2p = pl.pallas_call(
        _tc_b_body,
        grid=(NPAD // RB,),
        in_specs=[pl.BlockSpec((2, PB, PK), lambda i: (0, i, 0)),
                  pl.BlockSpec((PB, PK), lambda i: (i, 0)),
                  pl.BlockSpec((PB, PK), lambda i: (i, 0)),
                  pl.BlockSpec((1, PK), lambda i: (0, 0)),
                  pl.BlockSpec((HID, HID), lambda i: (0, 0))],
        out_specs=(pl.BlockSpec((PB, PK), lambda i: (i, 0)),
                   pl.BlockSpec((PB, PK), lambda i: (i, 0))),
        out_shape=(jax.ShapeDtypeStruct((PR, PK), f32),
                   jax.ShapeDtypeStruct((PR, PK), f32)),
    )(s1.reshape(2, PR, PK), h1p, dvp, jnp.tile(b1, 8).reshape(1, PK), W2)

    s2 = _scatter_kernel(g2p.reshape(NPAD, HID), ei_r)
    t2 = t2p.reshape(NPAD, HID)

    BR = ND // 8
    out = pl.pallas_call(
        _tc_c_body,
        grid=(8,),
        in_specs=[pl.BlockSpec((2, BR, HID), lambda i: (0, i, 0)),
                  pl.BlockSpec((BR, HID), lambda i: (i, 0)),
                  pl.BlockSpec((2, BR), lambda i: (0, i)),
                  pl.BlockSpec((2, ND, HID), lambda i: (0, 0, 0)),
                  pl.BlockSpec((ND, HID), lambda i: (0, 0)),
                  pl.BlockSpec((2, ND), lambda i: (0, 0)),
                  pl.BlockSpec((1, HID), lambda i: (0, 0)),
                  pl.BlockSpec((HID, HID), lambda i: (0, 0))],
        out_specs=pl.BlockSpec((BR, ND), lambda i: (i, 0)),
        out_shape=jax.ShapeDtypeStruct((ND, ND), f32),
    )(s2, t2, degp, s2, t2, degp, b2.reshape(1, HID), predictor)
    return out


# speedup vs baseline: 85.8130x; 1.0574x over previous
import functools

import jax
import jax.numpy as jnp
from jax import lax
from jax.experimental import pallas as pl
from jax.experimental.pallas import tpu as pltpu
from jax.experimental.pallas import tpu_sc as plsc

N = 10000
NPAD = 10240
E = 320000
D_FEAT = 128
HID = 16
ND = 2048
NW = 32
C = 128
ROWS = E // C
CH = ROWS // NW
XTRA = ROWS - CH * NW
SEG = NPAD // 16
NB = 12
HF = NB // 2
CHM = (CH // NB) * NB

_sc_mesh = plsc.VectorSubcoreMesh(core_axis_name="c", subcore_axis_name="s")
_sc_params = pltpu.CompilerParams(use_tc_tiling_on_sc=False)


@functools.partial(
    pl.kernel,
    out_type=jax.ShapeDtypeStruct((2, NPAD), jnp.float32),
    mesh=_sc_mesh,
    compiler_params=_sc_params,
    scratch_types=[
        pltpu.VMEM((CH, C), jnp.int32),
        pltpu.VMEM((XTRA, C), jnp.int32),
        pltpu.VMEM((C,), jnp.float32),
        pltpu.VMEM((SEG,), jnp.float32),
        pltpu.VMEM_SHARED((NPAD,), jnp.float32),
        pltpu.SemaphoreType.DMA,
    ],
)
def _deg_kernel(ei_hbm, out_hbm, dst_v, xdst_v, ones_v, tmp_v, acc, sem):
    cid = lax.axis_index("c")
    sid = lax.axis_index("s")
    wid = sid * 2 + cid
    base = wid * CH

    def _ones(i, _):
        ones_v[pl.ds(i * 16, 16)] = jnp.ones((16,), jnp.float32)
        return 0
    lax.fori_loop(0, C // 16, _ones, 0)

    def _zero(i, _):
        tmp_v[pl.ds(i * 16, 16)] = jnp.zeros((16,), jnp.float32)
        return 0
    lax.fori_loop(0, SEG // 16, _zero, 0)
    pltpu.sync_copy(tmp_v, acc.at[pl.ds(sid * SEG, SEG)])
    pltpu.sync_copy(ei_hbm.at[1, pl.ds(base, CH)], dst_v)

    @pl.when(wid == NW - 1)
    def _():
        pltpu.sync_copy(ei_hbm.at[1, pl.ds(CH * NW, XTRA)], xdst_v)
    plsc.subcore_barrier()

    DG = 6
    def _grp(i, _):
        for b in range(DG):
            pltpu.async_copy(ones_v, acc.at[dst_v.at[DG * i + b]], sem,
                             add=True)
        for b in range(DG):
            pltpu.make_async_copy(ones_v, acc.at[dst_v.at[0]], sem).wait()
        return 0
    lax.fori_loop(0, CH // DG, _grp, 0)

    @pl.when(wid == NW - 1)
    def _():
        for k in range(XTRA):
            pltpu.async_copy(ones_v, acc.at[xdst_v.at[k]], sem, add=True)
        for k in range(XTRA):
            pltpu.make_async_copy(ones_v, acc.at[xdst_v.at[0]], sem).wait()
    plsc.subcore_barrier()

    pltpu.sync_copy(acc.at[pl.ds(sid * SEG, SEG)], tmp_v)
    pltpu.sync_copy(tmp_v, out_hbm.at[cid, pl.ds(sid * SEG, SEG)])


@functools.partial(
    pl.kernel,
    out_type=jax.ShapeDtypeStruct((2, NPAD, HID), jnp.float32),
    mesh=_sc_mesh,
    compiler_params=_sc_params,
    scratch_types=[
        pltpu.VMEM((CH, C), jnp.int32),
        pltpu.VMEM((CH, C), jnp.int32),
        pltpu.VMEM((XTRA, C), jnp.int32),
        pltpu.VMEM((XTRA, C), jnp.int32),
        pltpu.VMEM((NB, C, HID), jnp.float32),
        pltpu.VMEM((SEG, HID), jnp.float32),
        pltpu.VMEM_SHARED((NPAD, HID), jnp.float32),
        pltpu.SemaphoreType.DMA((NB,)),
        pltpu.SemaphoreType.DMA((NB,)),
    ],
)
def _scatter_kernel(g_hbm, ei_hbm, out_hbm, src_v, dst_v, xsrc_v, xdst_v,
                    rows_v, tmp_v, acc, sem_g, sem_s):
    cid = lax.axis_index("c")
    sid = lax.axis_index("s")
    wid = sid * 2 + cid
    base = wid * CH

    def _zero(i, _):
        tmp_v[i, :] = jnp.zeros((16,), jnp.float32)
        return 0
    lax.fori_loop(0, C, _zero, 0)
    for k in range(SEG // C):
        pltpu.sync_copy(tmp_v.at[pl.ds(0, C)],
                        acc.at[pl.ds(sid * SEG + k * C, C)])
    pltpu.sync_copy(ei_hbm.at[0, pl.ds(base, CH)], src_v)
    pltpu.sync_copy(ei_hbm.at[1, pl.ds(base, CH)], dst_v)

    @pl.when(wid == NW - 1)
    def _():
        pltpu.sync_copy(ei_hbm.at[0, pl.ds(CH * NW, XTRA)], xsrc_v)
        pltpu.sync_copy(ei_hbm.at[1, pl.ds(CH * NW, XTRA)], xdst_v)
    plsc.subcore_barrier()

    def _wait_g(b):
        pltpu.make_async_copy(g_hbm.at[src_v.at[0]], rows_v.at[b],
                              sem_g.at[b]).wait()

    def _wait_s(b):
        pltpu.make_async_copy(rows_v.at[b], acc.at[dst_v.at[0]],
                              sem_s.at[b]).wait()

    for b in range(HF):
        pltpu.async_copy(g_hbm.at[src_v.at[b]], rows_v.at[b], sem_g.at[b])

    def _grp(i, _):
        for b in range(NB):
            c = NB * i + b
            bb = (b + HF) % NB
            _wait_g(b)
            pltpu.async_copy(rows_v.at[b], acc.at[dst_v.at[c]],
                             sem_s.at[b], add=True)

            @pl.when(c >= HF)
            def _():
                _wait_s(bb)

            @pl.when(c + HF < CH)
            def _():
                pltpu.async_copy(g_hbm.at[src_v.at[c + HF]], rows_v.at[bb],
                                 sem_g.at[bb])
        return 0
    lax.fori_loop(0, CHM // NB, _grp, 0)

    for c in range(CHM, CH):
        b = c % NB
        bb = (b + HF) % NB
        _wait_g(b)
        pltpu.async_copy(rows_v.at[b], acc.at[dst_v.at[c]],
                         sem_s.at[b], add=True)
        _wait_s(bb)
        if c + HF < CH:
            pltpu.async_copy(g_hbm.at[src_v.at[c + HF]], rows_v.at[bb],
                             sem_g.at[bb])
    @pl.when(wid == NW - 1)
    def _():
        for k in range(XTRA):
            pltpu.async_copy(g_hbm.at[xsrc_v.at[k]], rows_v.at[6 + k],
                             sem_g.at[6 + k])
        for k in range(XTRA):
            pltpu.make_async_copy(g_hbm.at[xsrc_v.at[0]], rows_v.at[6 + k],
                                  sem_g.at[6 + k]).wait()
            pltpu.async_copy(rows_v.at[6 + k], acc.at[xdst_v.at[k]],
                             sem_s.at[6 + k], add=True)
        for b in range(6 + XTRA):
            _wait_s(b)

    @pl.when(wid != NW - 1)
    def _():
        for b in range(6):
            _wait_s(b)
    plsc.subcore_barrier()

    pltpu.sync_copy(acc.at[pl.ds(sid * SEG, SEG)], tmp_v)
    pltpu.sync_copy(tmp_v, out_hbm.at[cid, pl.ds(sid * SEG, SEG)])


def _tc_a_body(xp_ref, w1_ref, deg8_ref, rep_ref, h_ref, g_ref, dv_ref):
    d8 = deg8_ref[0] + deg8_ref[1] + 1.0
    dv = lax.rsqrt(jnp.dot(d8, rep_ref[...],
                           preferred_element_type=jnp.float32))
    xp = xp_ref[...]
    w1 = w1_ref[...]
    h = jnp.concatenate(
        [jnp.dot(xp[:, 128 * a:128 * (a + 1)], w1,
                 preferred_element_type=jnp.float32) for a in range(8)],
        axis=1)
    h_ref[...] = h
    g_ref[...] = h * dv
    dv_ref[...] = dv


def _tc_b_body(s1_ref, h_ref, dv_ref, b1t_ref, w2_ref, t2_ref, g2_ref):
    dv = dv_ref[...]
    w2 = w2_ref[...]
    s = s1_ref[0] + s1_ref[1]
    h1 = jnp.maximum(
        s * dv + h_ref[...] * dv * dv + b1t_ref[0, :][None, :], 0.0)
    t2 = jnp.concatenate(
        [jnp.dot(h1[:, 16 * a:16 * (a + 1)], w2,
                 preferred_element_type=jnp.float32) for a in range(8)],
        axis=1)
    t2_ref[...] = t2
    g2_ref[...] = t2 * dv


def _h2_rows(s2, t2, degp, b2):
    deg = degp[0, :] + degp[1, :] + 1.0
    dinv = lax.rsqrt(deg)
    return (s2[0] + s2[1]) * dinv[:, None] \
        + t2 * (dinv * dinv)[:, None] + b2[0, :][None, :]


def _tc_c_body(s2b, t2b, dgb, s2f, t2f, dgf, b2_ref, pred_ref, o_ref):
    b2 = b2_ref[...]
    d_i = _h2_rows(s2b[...], t2b[...], dgb[...], b2)
    d_f = _h2_rows(s2f[...], t2f[...], dgf[...], b2)
    dp_i = jnp.dot(d_i, pred_ref[...], preferred_element_type=jnp.float32)
    o_ref[...] = lax.dot_general(
        dp_i, d_f, (((1,), (1,)), ((), ())),
        preferred_element_type=jnp.float32)


def kernel(x, edge_index, W1, b1, W2, b2, predictor, number_of_drugs):
    f32 = jnp.float32
    ei_r = edge_index.reshape(2, ROWS, C)

    degp = _deg_kernel(ei_r)

    x_pad = jnp.pad(x, ((0, NPAD - N), (0, 0)))
    RB = 2048
    PR = NPAD // 8
    PB = RB // 8
    PK = 128
    xp = x_pad.reshape(PR, 8 * D_FEAT)
    rep = jnp.repeat(jnp.eye(8, dtype=f32), HID, axis=1)
    deg8 = degp.reshape(2, PR, 8)
    h1p, g1p, dvp = pl.pallas_call(
        _tc_a_body,
        grid=(NPAD // RB,),
        in_specs=[pl.BlockSpec((PB, 8 * D_FEAT), lambda i: (i, 0)),
                  pl.BlockSpec((D_FEAT, HID), lambda i: (0, 0)),
                  pl.BlockSpec((2, PB, 8), lambda i: (0, i, 0)),
                  pl.BlockSpec((8, PK), lambda i: (0, 0))],
        out_specs=(pl.BlockSpec((PB, PK), lambda i: (i, 0)),
                   pl.BlockSpec((PB, PK), lambda i: (i, 0)),
                   pl.BlockSpec((PB, PK), lambda i: (i, 0))),
        out_shape=(jax.ShapeDtypeStruct((PR, PK), f32),
                   jax.ShapeDtypeStruct((PR, PK), f32),
                   jax.ShapeDtypeStruct((PR, PK), f32)),
    )(xp, W1, deg8, rep)

    s1 = _scatter_kernel(g1p.reshape(NPAD, HID), ei_r)

    t2p, g2p = pl.pallas_call(
        _tc_b_body,
        grid=(NPAD // RB,),
        in_specs=[pl.BlockSpec((2, PB, PK), lambda i: (0, i, 0)),
                  pl.BlockSpec((PB, PK), lambda i: (i, 0)),
                  pl.BlockSpec((PB, PK), lambda i: (i, 0)),
                  pl.BlockSpec((1, PK), lambda i: (0, 0)),
                  pl.BlockSpec((HID, HID), lambda i: (0, 0))],
        out_specs=(pl.BlockSpec((PB, PK), lambda i: (i, 0)),
                   pl.BlockSpec((PB, PK), lambda i: (i, 0))),
        out_shape=(jax.ShapeDtypeStruct((PR, PK), f32),
                   jax.ShapeDtypeStruct((PR, PK), f32)),
    )(s1.reshape(2, PR, PK), h1p, dvp, jnp.tile(b1, 8).reshape(1, PK), W2)

    s2 = _scatter_kernel(g2p.reshape(NPAD, HID), ei_r)
    s2s = s2[:, :ND, :]
    t2s = t2p[:ND // 8].reshape(ND, HID)
    degs = degp[:, :ND]

    BR = ND // 8
    out = pl.pallas_call(
        _tc_c_body,
        grid=(8,),
        in_specs=[pl.BlockSpec((2, BR, HID), lambda i: (0, i, 0)),
                  pl.BlockSpec((BR, HID), lambda i: (i, 0)),
                  pl.BlockSpec((2, BR), lambda i: (0, i)),
                  pl.BlockSpec((2, ND, HID), lambda i: (0, 0, 0)),
                  pl.BlockSpec((ND, HID), lambda i: (0, 0)),
                  pl.BlockSpec((2, ND), lambda i: (0, 0)),
                  pl.BlockSpec((1, HID), lambda i: (0, 0)),
                  pl.BlockSpec((HID, HID), lambda i: (0, 0))],
        out_specs=pl.BlockSpec((BR, ND), lambda i: (i, 0)),
        out_shape=jax.ShapeDtypeStruct((ND, ND), f32),
    )(s2s, t2s, degs, s2s, t2s, degs, b2.reshape(1, HID), predictor)
    return out
